# baseline (speedup 1.0000x reference)
import functools

import jax
import jax.numpy as jnp
from jax import lax
from jax.experimental import pallas as pl
from jax.experimental.pallas import tpu as pltpu
from jax.experimental.pallas import tpu_sc as plsc

B = 1024
S = 200
M = 64
D = 64
NC = 2
NS = 16
NW = NC * NS
R = B // NW

S_SPLIT = (104, 96)


def _sort_body(lr_ref, lc_ref, rank_ref, idxsort_ref, lsorted_ref):
    lr = lr_ref[...]
    lc = lc_ref[...]
    ii = lax.broadcasted_iota(jnp.int32, (B, B), 0)
    jj = lax.broadcasted_iota(jnp.int32, (B, B), 1)
    before = (lr > lc) | ((lr == lc) & (jj < ii))
    rank = jnp.sum(before.astype(jnp.int32), axis=1, keepdims=True)
    rank_ref[...] = rank
    onehot = rank == jj
    idxsort_ref[...] = jnp.sum(jnp.where(onehot, ii, 0), axis=0, keepdims=True)
    lsorted_ref[...] = jnp.sum(jnp.where(onehot, lc, 0), axis=0, keepdims=True)


def _sort_tc(length_flat):
    lr = length_flat.reshape(1, B)
    lc = length_flat.reshape(B, 1)
    rank, idx_sort, lsorted = pl.pallas_call(
        _sort_body,
        out_shape=[
            jax.ShapeDtypeStruct((B, 1), jnp.int32),
            jax.ShapeDtypeStruct((1, B), jnp.int32),
            jax.ShapeDtypeStruct((1, B), jnp.int32),
        ],
    )(lr, lc)
    return idx_sort.reshape(B), rank.reshape(B), lsorted.reshape(B)


def _sc_body(idx_hbm, idx8_hbm, ct_hbm, mat_hbm, tab_hbm, out_hbm,
             idxs_v, idx8_v, ct_v, mrow_v, trow_v, semg, semt):
    c = lax.axis_index("c")
    s = lax.axis_index("s")
    wid = s * NC + c
    base = wid * R
    pltpu.sync_copy(idx_hbm.at[pl.ds(base, R)], idxs_v)
    pltpu.sync_copy(idx8_hbm.at[pl.ds(base, R)], idx8_v)
    pltpu.async_copy(ct_hbm.at[idxs_v], ct_v, semg).wait()

    def row(r, carry):
        k = base + r
        pltpu.async_copy(mat_hbm.at[idx8_v.at[r, pl.ds(0, 1)]], mrow_v, semg)
        o0, n0 = 0, S_SPLIT[0]
        o1, n1 = S_SPLIT[0], S_SPLIT[1]
        pltpu.async_copy(tab_hbm.at[ct_v.at[r, pl.ds(o0, n0)]],
                         trow_v.at[pl.ds(o0, n0)], semt)
        pltpu.async_copy(tab_hbm.at[ct_v.at[r, pl.ds(o1, n1)]],
                         trow_v.at[pl.ds(o1, n1)], semt)
        pltpu.make_async_copy(mat_hbm.at[idx8_v.at[r, pl.ds(0, 1)]], mrow_v, semg).wait()
        pltpu.make_async_copy(tab_hbm.at[ct_v.at[r, pl.ds(o0, n0)]],
                              trow_v.at[pl.ds(o0, n0)], semt).wait()
        pltpu.make_async_copy(tab_hbm.at[ct_v.at[r, pl.ds(o1, n1)]],
                              trow_v.at[pl.ds(o1, n1)], semt).wait()
        pltpu.sync_copy(mrow_v.at[0], out_hbm.at[k, :, pl.ds(0, M)])
        pltpu.sync_copy(trow_v, out_hbm.at[k, :, pl.ds(M, D)])
        return carry

    lax.fori_loop(0, R, row, 0)


@jax.jit
def _run(matrix, length, core_terms, table):
    length_flat = length.reshape(B)
    idx_sort, idx_unsort, length_sorted = _sort_tc(length_flat)

    idx8 = jnp.pad(idx_sort.reshape(B, 1), ((0, 0), (0, 7)))
    ct = core_terms.reshape(B, S)
    mat = matrix.reshape(B, S, M)

    mesh = plsc.VectorSubcoreMesh(core_axis_name="c", subcore_axis_name="s")
    x = pl.kernel(
        _sc_body,
        mesh=mesh,
        compiler_params=pltpu.CompilerParams(use_tc_tiling_on_sc=False),
        out_type=jax.ShapeDtypeStruct((B, S, M + D), jnp.float32),
        scratch_types=[
            pltpu.VMEM((R,), jnp.int32),
            pltpu.VMEM((R, 8), jnp.int32),
            pltpu.VMEM((R, S), jnp.int32),
            pltpu.VMEM((1, S, M), jnp.float32),
            pltpu.VMEM((S, D), jnp.float32),
            pltpu.SemaphoreType.DMA,
            pltpu.SemaphoreType.DMA,
        ],
    )(idx_sort, idx8, ct, mat, table)
    return x, length_sorted, idx_unsort


def kernel(matrix, length, core_terms, table):
    return _run(matrix, length, core_terms, table)
